# baseline (speedup 1.0000x reference)
import jax
import jax.numpy as jnp
from jax import lax
from jax.experimental import pallas as pl
from jax.experimental.pallas import tpu as pltpu
from jax.experimental.pallas import tpu_sc as plsc

NSN = 10000
NTN = 10000
TOT = NSN + NTN
HID = 128
EDG = 320000
NB = 64
NLAYER = 3
DEA = 16

NC = 2
NSUB = 16
K = 128
NROW = EDG // K
NCH = NROW // NSUB
NXTRA = NROW - NCH * NSUB
RPT = 640
NPAD = NSUB * RPT

_sc_mesh = plsc.VectorSubcoreMesh(
    core_axis_name="c", subcore_axis_name="s", num_cores=NC, num_subcores=NSUB)


def _spmm_body(hpair_hbm, eidx_hbm, zg_hbm, out_hbm,
               acc, ib00, ib01, ib10, ib11, rb0, rb1,
               is00, is01, is10, is11, gsem0, gsem1, ssem0, ssem1):
    c = lax.axis_index("c")
    s = lax.axis_index("s")
    pltpu.sync_copy(zg_hbm, acc.at[pl.ds(s * RPT, RPT)])
    plsc.subcore_barrier()
    base = s * NCH
    htab = hpair_hbm.at[1 - c]

    def fetch(j, ib, isem):
        pltpu.async_copy(eidx_hbm.at[c, pl.ds((base + j) * K, K)], ib.at[0], isem)
        pltpu.async_copy(eidx_hbm.at[1 - c, pl.ds((base + j) * K, K)], ib.at[1], isem)

    def wfetch(ib, isem):
        pltpu.make_async_copy(eidx_hbm.at[0, pl.ds(0, K)], ib.at[0], isem).wait()
        pltpu.make_async_copy(eidx_hbm.at[0, pl.ds(0, K)], ib.at[1], isem).wait()

    def gath(ib, rb, gsem):
        pltpu.async_copy(htab.at[ib.at[0]], rb, gsem)

    def wgath(ib, rb, gsem):
        pltpu.make_async_copy(htab.at[ib.at[0]], rb, gsem).wait()

    def scat(ib, rb, ssem):
        pltpu.async_copy(rb, acc.at[ib.at[1]], ssem, add=True)

    def wscat(ib, rb, ssem):
        pltpu.make_async_copy(rb, acc.at[ib.at[1]], ssem).wait()

    fetch(0, ib00, is00)
    fetch(1, ib10, is10)
    fetch(2, ib01, is01)
    fetch(3, ib11, is11)
    wfetch(ib00, is00)
    gath(ib00, rb0, gsem0)
    wfetch(ib10, is10)
    gath(ib10, rb1, gsem1)

    def quad(g, carry):
        j0 = g * 4
        wgath(ib00, rb0, gsem0)
        scat(ib00, rb0, ssem0)
        wgath(ib10, rb1, gsem1)
        scat(ib10, rb1, ssem1)
        wscat(ib00, rb0, ssem0)
        fetch(j0 + 4, ib00, is00)
        wfetch(ib01, is01)
        gath(ib01, rb0, gsem0)
        wscat(ib10, rb1, ssem1)
        fetch(j0 + 5, ib10, is10)
        wfetch(ib11, is11)
        gath(ib11, rb1, gsem1)

        wgath(ib01, rb0, gsem0)
        scat(ib01, rb0, ssem0)
        wgath(ib11, rb1, gsem1)
        scat(ib11, rb1, ssem1)
        wscat(ib01, rb0, ssem0)
        fetch(j0 + 6, ib01, is01)
        wfetch(ib00, is00)
        gath(ib00, rb0, gsem0)
        wscat(ib11, rb1, ssem1)
        fetch(j0 + 7, ib11, is11)
        wfetch(ib10, is10)
        gath(ib10, rb1, gsem1)
        return carry

    lax.fori_loop(0, (NCH - 4) // 4, quad, 0)
    wgath(ib00, rb0, gsem0)
    scat(ib00, rb0, ssem0)
    wgath(ib10, rb1, gsem1)
    scat(ib10, rb1, ssem1)
    wscat(ib00, rb0, ssem0)
    wfetch(ib01, is01)
    gath(ib01, rb0, gsem0)
    wscat(ib10, rb1, ssem1)
    wfetch(ib11, is11)
    gath(ib11, rb1, gsem1)
    wgath(ib01, rb0, gsem0)
    scat(ib01, rb0, ssem0)
    wgath(ib11, rb1, gsem1)
    scat(ib11, rb1, ssem1)
    wscat(ib01, rb0, ssem0)
    wscat(ib11, rb1, ssem1)

    @pl.when(s < NXTRA)
    def _():
        jx = (NCH * NSUB - s * NCH) + s
        fetch(jx, ib00, is00)
        wfetch(ib00, is00)
        gath(ib00, rb0, gsem0)
        wgath(ib00, rb0, gsem0)
        scat(ib00, rb0, ssem0)
        wscat(ib00, rb0, ssem0)

    plsc.subcore_barrier()
    pltpu.sync_copy(acc.at[pl.ds(s * RPT, RPT)],
                    out_hbm.at[pl.ds(c * NPAD + s * RPT, RPT)])


_spmm_call = pl.kernel(
    _spmm_body,
    out_type=jax.ShapeDtypeStruct((2 * NPAD, HID), jnp.float32),
    mesh=_sc_mesh,
    compiler_params=pltpu.CompilerParams(use_tc_tiling_on_sc=False),
    scratch_types=[
        pltpu.VMEM_SHARED((NPAD, HID), jnp.float32),
        pltpu.VMEM((2, K), jnp.int32),
        pltpu.VMEM((2, K), jnp.int32),
        pltpu.VMEM((2, K), jnp.int32),
        pltpu.VMEM((2, K), jnp.int32),
        pltpu.VMEM((K, HID), jnp.float32),
        pltpu.VMEM((K, HID), jnp.float32),
        pltpu.SemaphoreType.DMA,
        pltpu.SemaphoreType.DMA,
        pltpu.SemaphoreType.DMA,
        pltpu.SemaphoreType.DMA,
        pltpu.SemaphoreType.DMA,
        pltpu.SemaphoreType.DMA,
        pltpu.SemaphoreType.DMA,
        pltpu.SemaphoreType.DMA,
    ],
)


def _ea_body(ea_hbm, eidx_hbm, zea_hbm, oea_hbm, ocnt_hbm,
             eacc, cacc, ib0, ib1, eb0, eb1, ones,
             is0, is1, vs0, vs1, esem0, esem1, csem0, csem1):
    c = lax.axis_index("c")
    s = lax.axis_index("s")
    pltpu.sync_copy(zea_hbm, eacc.at[pl.ds(s * RPT, RPT)])
    pltpu.sync_copy(zea_hbm, cacc.at[pl.ds(s * RPT, RPT)])
    one_row = jnp.zeros((16,), jnp.float32) + 1.0

    def fill(r, carry):
        ones[r, pl.ds(0, 16)] = one_row
        return carry

    lax.fori_loop(0, K, fill, 0)
    plsc.subcore_barrier()
    base = s * NCH
    vbase = s * (NCH * K)

    def fetch(j, ib, isem):
        pltpu.async_copy(eidx_hbm.at[1 - c, pl.ds((base + j) * K, K)], ib, isem)

    def wfetch(ib, isem):
        pltpu.make_async_copy(eidx_hbm.at[0, pl.ds(0, K)], ib, isem).wait()

    def vload(j, eb, vsem):
        pltpu.async_copy(ea_hbm.at[pl.ds(vbase + j * K, K)], eb, vsem)

    def wvload(eb, vsem):
        pltpu.make_async_copy(ea_hbm.at[pl.ds(0, K)], eb, vsem).wait()

    def scat(ib, eb, esem, csem):
        pltpu.async_copy(eb, eacc.at[ib], esem, add=True)
        pltpu.async_copy(ones, cacc.at[ib], csem, add=True)

    def wscat(ib, eb, esem, csem):
        pltpu.make_async_copy(eb, eacc.at[ib], esem).wait()
        pltpu.make_async_copy(ones, cacc.at[ib], csem).wait()

    fetch(0, ib0, is0)
    fetch(1, ib1, is1)
    vload(0, eb0, vs0)
    vload(1, eb1, vs1)

    def pair(g, carry):
        j0 = g * 2
        wfetch(ib0, is0)
        wvload(eb0, vs0)
        scat(ib0, eb0, esem0, csem0)
        wfetch(ib1, is1)
        wvload(eb1, vs1)
        scat(ib1, eb1, esem1, csem1)
        wscat(ib0, eb0, esem0, csem0)
        fetch(j0 + 2, ib0, is0)
        vload(j0 + 2, eb0, vs0)
        wscat(ib1, eb1, esem1, csem1)
        fetch(j0 + 3, ib1, is1)
        vload(j0 + 3, eb1, vs1)
        return carry

    lax.fori_loop(0, (NCH - 2) // 2, pair, 0)
    wfetch(ib0, is0)
    wvload(eb0, vs0)
    scat(ib0, eb0, esem0, csem0)
    wfetch(ib1, is1)
    wvload(eb1, vs1)
    scat(ib1, eb1, esem1, csem1)
    wscat(ib0, eb0, esem0, csem0)
    wscat(ib1, eb1, esem1, csem1)

    @pl.when(s < NXTRA)
    def _():
        jx = (NCH * NSUB - s * NCH) + s
        fetch(jx, ib0, is0)
        vload((NCH * NSUB + s) - s * NCH, eb0, vs0)
        wfetch(ib0, is0)
        wvload(eb0, vs0)
        scat(ib0, eb0, esem0, csem0)
        wscat(ib0, eb0, esem0, csem0)

    plsc.subcore_barrier()
    pltpu.sync_copy(eacc.at[pl.ds(s * RPT, RPT)],
                    oea_hbm.at[pl.ds(c * NPAD + s * RPT, RPT)])
    pltpu.sync_copy(cacc.at[pl.ds(s * RPT, RPT)],
                    ocnt_hbm.at[pl.ds(c * NPAD + s * RPT, RPT)])


_ea_call = pl.kernel(
    _ea_body,
    out_type=(jax.ShapeDtypeStruct((2 * NPAD, DEA), jnp.float32),
              jax.ShapeDtypeStruct((2 * NPAD, DEA), jnp.float32)),
    mesh=_sc_mesh,
    compiler_params=pltpu.CompilerParams(use_tc_tiling_on_sc=False),
    scratch_types=[
        pltpu.VMEM_SHARED((NPAD, DEA), jnp.float32),
        pltpu.VMEM_SHARED((NPAD, DEA), jnp.float32),
        pltpu.VMEM((K,), jnp.int32),
        pltpu.VMEM((K,), jnp.int32),
        pltpu.VMEM((K, DEA), jnp.float32),
        pltpu.VMEM((K, DEA), jnp.float32),
        pltpu.VMEM((K, DEA), jnp.float32),
        pltpu.SemaphoreType.DMA,
        pltpu.SemaphoreType.DMA,
        pltpu.SemaphoreType.DMA,
        pltpu.SemaphoreType.DMA,
        pltpu.SemaphoreType.DMA,
        pltpu.SemaphoreType.DMA,
        pltpu.SemaphoreType.DMA,
        pltpu.SemaphoreType.DMA,
    ],
)


RBLK = 2000
GRID = NSN // RBLK


def _layer_body(h2, g2, ea2, cnt2, bids, wx, we, wself, bias,
                hout_ref, pool_ref):
    f32 = jnp.float32
    i = pl.program_id(1)
    iot = lax.broadcasted_iota(jnp.int32, (1, NB), 1)

    @pl.when(i == 0)
    def _():
        pool_ref[...] = jnp.zeros(pool_ref.shape, f32)

    inv = 1.0 / jnp.maximum(cnt2[0][:, 0:1], 1.0)
    agg = (jnp.dot(g2[0], wx[0], preferred_element_type=f32)
           + jnp.dot(ea2[0], we[0], preferred_element_type=f32)) * inv
    nh = jnp.maximum(
        jnp.dot(h2[0], wself[0], preferred_element_type=f32)
        + agg + bias[0], 0.0)
    hout_ref[0] = nh
    mask = jnp.where(bids[0] == iot, 1.0, 0.0)
    pool_ref[0] += lax.dot_general(
        mask, nh, (((0,), (0,)), ((), ())), preferred_element_type=f32)


_layer_call = pl.pallas_call(
    _layer_body,
    grid=(2, GRID),
    in_specs=[
        pl.BlockSpec((1, RBLK, HID), lambda b, i: (b, i, 0)),
        pl.BlockSpec((1, RBLK, HID), lambda b, i: (b, i, 0)),
        pl.BlockSpec((1, RBLK, DEA), lambda b, i: (b, i, 0)),
        pl.BlockSpec((1, RBLK, DEA), lambda b, i: (b, i, 0)),
        pl.BlockSpec((1, RBLK, 1), lambda b, i: (b, i, 0)),
        pl.BlockSpec((1, HID, HID), lambda b, i: (b, 0, 0)),
        pl.BlockSpec((1, DEA, HID), lambda b, i: (b, 0, 0)),
        pl.BlockSpec((1, HID, HID), lambda b, i: (b, 0, 0)),
        pl.BlockSpec((1, 1, HID), lambda b, i: (b, 0, 0)),
    ],
    out_specs=[
        pl.BlockSpec((1, RBLK, HID), lambda b, i: (b, i, 0)),
        pl.BlockSpec((1, NB, HID), lambda b, i: (b, 0, 0)),
    ],
    out_shape=[
        jax.ShapeDtypeStruct((2, NSN, HID), jnp.float32),
        jax.ShapeDtypeStruct((2, NB, HID), jnp.float32),
    ],
)


def _head_body(p0, p1, p2, y, wcfg, bcfg, wp, bp, out_ref):
    f32 = jnp.float32
    yemb = jnp.dot(y[...], wcfg[...], preferred_element_type=f32) + bcfg[...]
    acc = jnp.dot(p0[1], wp[0:128, :], preferred_element_type=f32)
    acc += jnp.dot(p1[1], wp[128:256, :], preferred_element_type=f32)
    acc += jnp.dot(p2[1], wp[256:384, :], preferred_element_type=f32)
    acc += jnp.dot(p0[0], wp[384:512, :], preferred_element_type=f32)
    acc += jnp.dot(p1[0], wp[512:640, :], preferred_element_type=f32)
    acc += jnp.dot(p2[0], wp[640:768, :], preferred_element_type=f32)
    acc += jnp.dot(yemb, wp[768:784, :], preferred_element_type=f32)
    out_ref[...] = acc + bp[...]


_head_call = pl.pallas_call(
    _head_body,
    out_shape=jax.ShapeDtypeStruct((NB, 1), jnp.float32),
)


def kernel(x_s, x_t, edge_attr, edge_index, x_s_batch, x_t_batch, y, params):
    f32 = jnp.float32
    eidx = edge_index.astype(jnp.int32)
    zg = jnp.zeros((RPT, HID), f32)
    zea = jnp.zeros((RPT, DEA), f32)
    bids = jnp.stack([x_t_batch, x_s_batch]).astype(jnp.int32).reshape(2, NSN, 1)

    eao, cnto = _ea_call(edge_attr, eidx, zea)
    ea3 = eao.reshape(2, NPAD, DEA)
    cnt3 = cnto.reshape(2, NPAD, DEA)

    h2 = jnp.stack([x_t, x_s])
    pools = []
    for l in range(NLAYER):
        p = params['layer%d' % l]
        g = _spmm_call(h2, eidx, zg)
        h2, pool = _layer_call(
            h2, g.reshape(2, NPAD, HID), ea3, cnt3,
            bids,
            jnp.stack([p['Ws2t'], p['Wt2s']]),
            jnp.stack([p['We2t'], p['We2s']]),
            jnp.stack([p['Wt_self'], p['Ws_self']]),
            jnp.stack([p['bt'], p['bs']]).reshape(2, 1, HID))
        pools.append(pool)

    return _head_call(
        pools[0], pools[1], pools[2],
        y, params['W_cfg'], params['b_cfg'].reshape(1, 16),
        params['W_pred'], params['b_pred'].reshape(1, 1))
